# Initial kernel scaffold; baseline (speedup 1.0000x reference)
import jax
import jax.numpy as jnp
from jax.experimental import pallas as pl


def kernel(input_ids, token_table, position_table):
    raise NotImplementedError("write your pallas kernel here")


# speedup vs baseline: 1.0499x; 1.0499x over previous
import jax
import jax.numpy as jnp
from jax import lax
from jax.experimental import pallas as pl
from jax.experimental.pallas import tpu as pltpu
from jax.experimental.pallas import tpu_sc as plsc

VOCAB = 100000
MAX_POSITION = 2048
EMBED = 1024
BATCH = 4
SEQ = 2048

NC, NS = 2, 16
NW = NC * NS
PPW = SEQ // NW
HALF = PPW // 2
LANES = 16
VECS = EMBED // LANES


def _embed_body(ids_hbm, tok_hbm, pos_hbm, out_hbm, idx_v, pos_v, buf, sem):
    wid = lax.axis_index("s") * NC + lax.axis_index("c")
    pstart = wid * PPW
    pltpu.sync_copy(ids_hbm.at[wid], idx_v)
    pltpu.sync_copy(pos_hbm.at[pl.ds(pstart, PPW)], pos_v)
    for b in range(BATCH):
        for h in range(2):
            pltpu.async_copy(
                tok_hbm.at[idx_v.at[b, pl.ds(h * HALF, HALF)]], buf, sem
            ).wait()

            def add_row(r, _, h=h):
                for v in range(VECS):
                    col = v * LANES
                    vec = pos_v[h * HALF + r, pl.ds(col, LANES)]
                    plsc.addupdate(buf.at[r, pl.ds(col, LANES)], vec)
                return 0

            lax.fori_loop(0, HALF, add_row, 0)
            pltpu.sync_copy(
                buf, out_hbm.at[pl.ds(b * SEQ + pstart + h * HALF, HALF)]
            )


@jax.jit
def _embed(ids3, token_table, position_table):
    mesh = plsc.VectorSubcoreMesh(core_axis_name="c", subcore_axis_name="s")
    k = pl.kernel(
        _embed_body,
        out_type=jax.ShapeDtypeStruct((BATCH * SEQ, EMBED), jnp.float32),
        mesh=mesh,
        scratch_types=[
            pltpu.VMEM((BATCH, PPW), jnp.int32),
            pltpu.VMEM((PPW, EMBED), jnp.float32),
            pltpu.VMEM((HALF, EMBED), jnp.float32),
            pltpu.SemaphoreType.DMA,
        ],
    )
    return k(ids3, token_table, position_table)


def kernel(input_ids, token_table, position_table):
    ids3 = jnp.transpose(
        input_ids.astype(jnp.int32).reshape(BATCH, NW, PPW), (1, 0, 2)
    )
    out = _embed(ids3, token_table, position_table)
    return out.reshape(BATCH, SEQ, EMBED)


# speedup vs baseline: 1.2528x; 1.1933x over previous
import jax
import jax.numpy as jnp
from jax import lax
from jax.experimental import pallas as pl
from jax.experimental.pallas import tpu as pltpu
from jax.experimental.pallas import tpu_sc as plsc

VOCAB = 100000
MAX_POSITION = 2048
EMBED = 1024
BATCH = 4
SEQ = 2048

NC, NS = 2, 16
NW = NC * NS
PPW = SEQ // NW
HALF = PPW // 2
LANES = 16
VECS = EMBED // LANES


def _embed_body(ids_hbm, tok_hbm, pos_hbm, out_hbm,
                idx_v, pos_v, buf0, buf1, gs0, gs1, ws0, ws1):
    wid = lax.axis_index("s") * NC + lax.axis_index("c")
    pstart = wid * PPW
    bufs, gsems, wsems = (buf0, buf1), (gs0, gs1), (ws0, ws1)
    pltpu.sync_copy(ids_hbm.at[wid], idx_v)

    def gather(h, b, slot):
        return pltpu.async_copy(
            tok_hbm.at[idx_v.at[b, pl.ds(h * HALF, HALF)]],
            bufs[slot], gsems[slot])

    for h in range(2):
        pltpu.sync_copy(pos_hbm.at[pl.ds(pstart + h * HALF, HALF)], pos_v)
        gdesc = [None, None]
        wdesc = [None, None]
        gdesc[0] = gather(h, 0, 0)
        for b in range(BATCH):
            cur, nxt = b % 2, (b + 1) % 2
            if b + 1 < BATCH:
                if wdesc[nxt] is not None:
                    wdesc[nxt].wait()
                gdesc[nxt] = gather(h, b + 1, nxt)
            gdesc[cur].wait()

            def add_row(r, _, cur=cur):
                for v in range(VECS):
                    col = v * LANES
                    vec = pos_v[r, pl.ds(col, LANES)]
                    plsc.addupdate(bufs[cur].at[r, pl.ds(col, LANES)], vec)
                return 0

            lax.fori_loop(0, HALF, add_row, 0)
            wdesc[cur] = pltpu.async_copy(
                bufs[cur],
                out_hbm.at[pl.ds(b * SEQ + pstart + h * HALF, HALF)],
                wsems[cur])
        for d in wdesc:
            if d is not None:
                d.wait()


@jax.jit
def _embed(ids3, token_table, position_table):
    mesh = plsc.VectorSubcoreMesh(core_axis_name="c", subcore_axis_name="s")
    k = pl.kernel(
        _embed_body,
        out_type=jax.ShapeDtypeStruct((BATCH * SEQ, EMBED), jnp.float32),
        mesh=mesh,
        scratch_types=[
            pltpu.VMEM((BATCH, PPW), jnp.int32),
            pltpu.VMEM((HALF, EMBED), jnp.float32),
            pltpu.VMEM((HALF, EMBED), jnp.float32),
            pltpu.VMEM((HALF, EMBED), jnp.float32),
            pltpu.SemaphoreType.DMA,
            pltpu.SemaphoreType.DMA,
            pltpu.SemaphoreType.DMA,
            pltpu.SemaphoreType.DMA,
        ],
    )
    return k(ids3, token_table, position_table)


def kernel(input_ids, token_table, position_table):
    ids3 = jnp.transpose(
        input_ids.astype(jnp.int32).reshape(BATCH, NW, PPW), (1, 0, 2)
    )
    out = _embed(ids3, token_table, position_table)
    return out.reshape(BATCH, SEQ, EMBED)


# speedup vs baseline: 1.5006x; 1.1978x over previous
import jax
import jax.numpy as jnp
from jax import lax
from jax.experimental import pallas as pl
from jax.experimental.pallas import tpu as pltpu
from jax.experimental.pallas import tpu_sc as plsc

VOCAB = 100000
MAX_POSITION = 2048
EMBED = 1024
BATCH = 4
SEQ = 2048

NC, NS = 2, 16
NW = NC * NS
PPW = SEQ // NW
HALF = PPW // 2
QROWS = 16
NSTEP = (BATCH * SEQ) // NW // QROWS
LANES = 16
VECS = EMBED // LANES
NBUF = 4
NROUND = NSTEP // NBUF


def _embed_body(ids_hbm, tok_hbm, pos_hbm, out_hbm,
                idx_v, pos_v, b0, b1, b2, b3,
                gs0, gs1, gs2, gs3, ws0, ws1, ws2, ws3):
    wid = lax.axis_index("s") * NC + lax.axis_index("c")
    pstart = wid * PPW
    bufs = (b0, b1, b2, b3)
    gsems = (gs0, gs1, gs2, gs3)
    wsems = (ws0, ws1, ws2, ws3)
    pltpu.sync_copy(ids_hbm.at[wid], idx_v)
    pltpu.sync_copy(pos_hbm.at[pl.ds(pstart, HALF)], pos_v)

    def sparams(s):
        h = s // 8
        b = (s % 8) // 2
        qh = s % 2
        poff = h * HALF + qh * QROWS
        return b, poff, qh * QROWS

    def issue_gather(s, slot):
        b, poff, _ = sparams(s)
        return pltpu.async_copy(
            tok_hbm.at[idx_v.at[b, pl.ds(poff, QROWS)]],
            bufs[slot], gsems[slot])

    for s in range(NBUF - 1):
        issue_gather(s, s)

    def round_body(t, _):
        for j in range(NBUF):
            s = t * NBUF + j
            b, poff, prow = sparams(s)

            @pl.when(s == NSTEP // 2)
            def _():
                pltpu.sync_copy(
                    pos_hbm.at[pl.ds(pstart + HALF, HALF)], pos_v)

            pltpu.make_async_copy(
                tok_hbm.at[idx_v.at[b, pl.ds(poff, QROWS)]],
                bufs[j], gsems[j]).wait()

            @plsc.parallel_loop(0, QROWS, 1)
            def add_row(r, j=j, prow=prow):
                for v in range(VECS):
                    col = v * LANES
                    plsc.addupdate(bufs[j].at[r, pl.ds(col, LANES)],
                                   pos_v[prow + r, pl.ds(col, LANES)])

            pltpu.async_copy(
                bufs[j], out_hbm.at[pl.ds(b * SEQ + pstart + poff, QROWS)],
                wsems[j])

            nslot = (j + NBUF - 1) % NBUF
            @pl.when(s >= 1)
            def _():
                pltpu.make_async_copy(
                    bufs[nslot], out_hbm.at[pl.ds(0, QROWS)],
                    wsems[nslot]).wait()

            @pl.when(s + NBUF - 1 < NSTEP)
            def _():
                issue_gather(s + NBUF - 1, nslot)
        return 0

    lax.fori_loop(0, NROUND, round_body, 0)
    pltpu.make_async_copy(
        bufs[NBUF - 1], out_hbm.at[pl.ds(0, QROWS)], wsems[NBUF - 1]).wait()


@jax.jit
def _embed(ids3, token_table, position_table):
    mesh = plsc.VectorSubcoreMesh(core_axis_name="c", subcore_axis_name="s")
    k = pl.kernel(
        _embed_body,
        out_type=jax.ShapeDtypeStruct((BATCH * SEQ, EMBED), jnp.float32),
        mesh=mesh,
        scratch_types=[
            pltpu.VMEM((BATCH, PPW), jnp.int32),
            pltpu.VMEM((HALF, EMBED), jnp.float32),
            pltpu.VMEM((QROWS, EMBED), jnp.float32),
            pltpu.VMEM((QROWS, EMBED), jnp.float32),
            pltpu.VMEM((QROWS, EMBED), jnp.float32),
            pltpu.VMEM((QROWS, EMBED), jnp.float32),
            pltpu.SemaphoreType.DMA,
            pltpu.SemaphoreType.DMA,
            pltpu.SemaphoreType.DMA,
            pltpu.SemaphoreType.DMA,
            pltpu.SemaphoreType.DMA,
            pltpu.SemaphoreType.DMA,
            pltpu.SemaphoreType.DMA,
            pltpu.SemaphoreType.DMA,
        ],
    )
    return k(ids3, token_table, position_table)


def kernel(input_ids, token_table, position_table):
    ids3 = jnp.transpose(
        input_ids.astype(jnp.int32).reshape(BATCH, NW, PPW), (1, 0, 2)
    )
    out = _embed(ids3, token_table, position_table)
    return out.reshape(BATCH, SEQ, EMBED)
